# baseline (speedup 1.0000x reference)
import functools

import jax
import jax.numpy as jnp
import numpy as np
from jax import lax
from jax.experimental import pallas as pl
from jax.experimental.pallas import tpu as pltpu
from jax.experimental.pallas import tpu_sc as plsc

SN = 2
NBR = 16
_NC, _NS = 2, 16
_NW = _NC * _NS


def _rup(n, m):
    return ((n + m - 1) // m) * m


def _pad_rows(x, n, fill=0.0):
    if x.shape[0] == n:
        return x
    pad = jnp.full((n - x.shape[0],) + x.shape[1:], fill, x.dtype)
    return jnp.concatenate([x, pad], axis=0)


def _pad_lanes(x, n):
    return jnp.concatenate(
        [x, jnp.zeros((x.shape[0], n - x.shape[1]), x.dtype)], axis=1)


def _topk_body(K, nearest, t_ref, st_ref, o_ref):
    t = t_ref[...]
    st = st_ref[...]
    S = st.shape[1]
    inner = jnp.dot(t, st)
    q_t = jnp.sum(t * t, axis=1, keepdims=True)
    q_s = jnp.sum(st * st, axis=0, keepdims=True)
    if nearest:
        dist = (q_s + q_t) - 2.0 * inner
    else:
        dist = (-2.0 * inner + q_s) + q_t
    iota = lax.broadcasted_iota(jnp.int32, dist.shape, 1)
    cols = []
    for _ in range(K):
        m = jnp.min(dist, axis=1, keepdims=True)
        am = jnp.min(jnp.where(dist == m, iota, S), axis=1, keepdims=True)
        cols.append(am)
        dist = jnp.where(iota == am, jnp.inf, dist)
    o_ref[...] = jnp.concatenate(cols, axis=1)


def _topk(targets, sources, K, nearest=False, Tb=256):
    T0 = targets.shape[0]
    S = sources.shape[0]
    T = _rup(T0, Tb)
    tgt = _pad_rows(_pad_lanes(targets, 16), T, 1e30)
    st = _pad_lanes(sources, 16).T
    out = pl.pallas_call(
        functools.partial(_topk_body, K, nearest),
        grid=(T // Tb,),
        in_specs=[
            pl.BlockSpec((Tb, 16), lambda i: (i, 0)),
            pl.BlockSpec((16, S), lambda i: (0, 0)),
        ],
        out_specs=pl.BlockSpec((Tb, K), lambda i: (i, 0)),
        out_shape=jax.ShapeDtypeStruct((T, K), jnp.int32),
    )(tgt, st)
    return out[:T0]


def _sc_gather_padded(table, idx, chunk):
    V, D = table.shape
    B = idx.shape[0]
    bpw = B // _NW
    npair = bpw // (2 * chunk)
    mesh = plsc.VectorSubcoreMesh(core_axis_name="c", subcore_axis_name="s")

    @functools.partial(
        pl.kernel,
        mesh=mesh,
        out_type=jax.ShapeDtypeStruct((B, D), jnp.float32),
        scratch_types=[
            pltpu.VMEM((bpw,), jnp.int32),
            pltpu.VMEM((2, chunk, D), jnp.float32),
            pltpu.SemaphoreType.DMA,
            pltpu.SemaphoreType.DMA,
        ],
    )
    def k(table_hbm, idx_hbm, out_hbm, idx_v, rows_v, gsem, osem):
        wid = lax.axis_index("s") * _NC + lax.axis_index("c")
        base = wid * bpw
        pltpu.sync_copy(idx_hbm.at[pl.ds(base, bpw)], idx_v)

        def pair(j, carry):
            for b in range(2):
                i = 2 * j + b
                @pl.when(j > 0)
                def _():
                    pltpu.make_async_copy(
                        out_hbm.at[pl.ds(base, chunk)], rows_v.at[b], osem
                    ).wait()

                pltpu.async_copy(
                    table_hbm.at[idx_v.at[pl.ds(i * chunk, chunk)]],
                    rows_v.at[b], gsem,
                ).wait()
                pltpu.async_copy(
                    rows_v.at[b], out_hbm.at[pl.ds(base + i * chunk, chunk)],
                    osem,
                )
            return carry

        lax.fori_loop(0, npair, pair, 0)
        for b in range(2):
            pltpu.make_async_copy(
                out_hbm.at[pl.ds(base, chunk)], rows_v.at[b], osem
            ).wait()

    return k(table, idx)


def _sc_gather(table, idx):
    V, D = table.shape
    B0 = idx.shape[0]
    chunk = max(8, min(128, (110_000 // (2 * D)) // 8 * 8))
    quantum = _NW * chunk * 2
    B = _rup(B0, quantum)
    idx = _pad_rows(idx.astype(jnp.int32), B, 0)
    out = _sc_gather_padded(table, idx, chunk)
    return out[:B0]


def _mm_body(x_ref, w_ref, b_ref, o_ref):
    o_ref[...] = (
        jnp.dot(x_ref[...], w_ref[...], preferred_element_type=jnp.float32)
        + b_ref[...]
    )


def _matmul_bias(x, w, b, Rb):
    R, Cin = x.shape
    Cout = w.shape[1]
    return pl.pallas_call(
        _mm_body,
        grid=(R // Rb,),
        in_specs=[
            pl.BlockSpec((Rb, Cin), lambda i: (i, 0)),
            pl.BlockSpec((Cin, Cout), lambda i: (0, 0)),
            pl.BlockSpec((1, Cout), lambda i: (0, 0)),
        ],
        out_specs=pl.BlockSpec((Rb, Cout), lambda i: (i, 0)),
        out_shape=jax.ShapeDtypeStruct((R, Cout), jnp.float32),
    )(x, w, b[None, :])


def _theta(verts, nb, sd, Vb):
    diff = nb[:, :16].reshape(Vb, NBR, 16) - verts[:, None, :]
    ss = jnp.sum(diff * diff, axis=2, keepdims=True)
    nd = diff / jnp.maximum(jnp.sqrt(ss), 1e-12)
    th = jnp.dot(
        nd.reshape(Vb * NBR, 16), sd, preferred_element_type=jnp.float32
    )
    return jnp.maximum(th, 0.0)


def _convsurf_body(v_ref, nb_ref, sd_ref, o_ref):
    Vb = v_ref.shape[0]
    th = _theta(v_ref[...], nb_ref[...], sd_ref[...], Vb)
    m = jnp.max(th.reshape(Vb, NBR, th.shape[1]), axis=1)
    C = m.shape[1] // 2
    o_ref[...] = jnp.maximum(m[:, :C] + m[:, C:], 0.0)


def _conv_surface(vpad, nb, sd, Vb):
    V = vpad.shape[0]
    C = sd.shape[1] // 2
    return pl.pallas_call(
        _convsurf_body,
        grid=(V // Vb,),
        in_specs=[
            pl.BlockSpec((Vb, 16), lambda i: (i, 0)),
            pl.BlockSpec((Vb * NBR, 128), lambda i: (i, 0)),
            pl.BlockSpec((16, 2 * C), lambda i: (0, 0)),
        ],
        out_specs=pl.BlockSpec((Vb, C), lambda i: (i, 0)),
        out_shape=jax.ShapeDtypeStruct((V, C), jnp.float32),
    )(vpad, nb, sd)


def _convlayer_body(n_real, v_ref, nb_ref, sd_ref, fs_ref, fc_ref,
                    o_ref, s1_ref, s2_ref):
    Vb = v_ref.shape[0]
    th = _theta(v_ref[...], nb_ref[...], sd_ref[...], Vb)
    act = th * fs_ref[...]
    m = jnp.max(act.reshape(Vb, NBR, act.shape[1]), axis=1)
    C = m.shape[1] // 2
    y = fc_ref[...] + m[:, :C] + m[:, C:]
    o_ref[...] = y
    row = pl.program_id(0) * Vb + lax.broadcasted_iota(jnp.int32, (Vb, 1), 0)
    ym = jnp.where(row < n_real, y, 0.0)
    s1_ref[...] = jnp.sum(ym, axis=0).reshape(1, 1, C)
    s2_ref[...] = jnp.sum(ym * ym, axis=0).reshape(1, 1, C)


def _conv_layer(vpad, nb, sd, fs, fc, Vb, n_real):
    V = vpad.shape[0]
    C = fc.shape[1]
    nblk = V // Vb
    return pl.pallas_call(
        functools.partial(_convlayer_body, n_real),
        grid=(nblk,),
        in_specs=[
            pl.BlockSpec((Vb, 16), lambda i: (i, 0)),
            pl.BlockSpec((Vb * NBR, 128), lambda i: (i, 0)),
            pl.BlockSpec((16, 2 * C), lambda i: (0, 0)),
            pl.BlockSpec((Vb * NBR, 2 * C), lambda i: (i, 0)),
            pl.BlockSpec((Vb, C), lambda i: (i, 0)),
        ],
        out_specs=[
            pl.BlockSpec((Vb, C), lambda i: (i, 0)),
            pl.BlockSpec((1, 1, C), lambda i: (i, 0, 0)),
            pl.BlockSpec((1, 1, C), lambda i: (i, 0, 0)),
        ],
        out_shape=[
            jax.ShapeDtypeStruct((V, C), jnp.float32),
            jax.ShapeDtypeStruct((nblk, 1, C), jnp.float32),
            jax.ShapeDtypeStruct((nblk, 1, C), jnp.float32),
        ],
    )(vpad, nb, sd, fs, fc)


def _affine_body(x_ref, sc_ref, sh_ref, o_ref):
    o_ref[...] = jnp.maximum(x_ref[...] * sc_ref[...] + sh_ref[...], 0.0)


def _affine_relu(x, scale, shift, Rb):
    R, C = x.shape
    return pl.pallas_call(
        _affine_body,
        grid=(R // Rb,),
        in_specs=[
            pl.BlockSpec((Rb, C), lambda i: (i, 0)),
            pl.BlockSpec((1, C), lambda i: (0, 0)),
            pl.BlockSpec((1, C), lambda i: (0, 0)),
        ],
        out_specs=pl.BlockSpec((Rb, C), lambda i: (i, 0)),
        out_shape=jax.ShapeDtypeStruct((R, C), jnp.float32),
    )(x, scale[None, :], shift[None, :])


def _gmax_body(G, x_ref, o_ref):
    Vb = o_ref.shape[0]
    C = o_ref.shape[1]
    o_ref[...] = jnp.max(x_ref[...].reshape(Vb, G, C), axis=1)


def _group_max(x, G, Rb):
    RG, C = x.shape
    R = RG // G
    return pl.pallas_call(
        functools.partial(_gmax_body, G),
        grid=(R // Rb,),
        in_specs=[pl.BlockSpec((Rb * G, C), lambda i: (i, 0))],
        out_specs=pl.BlockSpec((Rb, C), lambda i: (i, 0)),
        out_shape=jax.ShapeDtypeStruct((R, C), jnp.float32),
    )(x)


def _sd(directions):
    n = jnp.linalg.norm(directions, axis=0, keepdims=True)
    sd = directions / jnp.maximum(n, 1e-12)
    return jnp.concatenate([sd, jnp.zeros((13, sd.shape[1]), sd.dtype)], axis=0)


def _bn_coeffs(s1, s2, n, gamma, beta, eps=1e-5):
    mean = jnp.sum(s1, axis=(0, 1)) / n
    var = jnp.sum(s2, axis=(0, 1)) / n - mean * mean
    scale = gamma / jnp.sqrt(var + eps)
    shift = beta - mean * scale
    return scale, shift


def kernel(vertices, dir0, w1, b1, dir1, w2, b2, dir2, w3, b3, dir3,
           w4, b4, dir4, g1, be1, g2, be2, g3, be3):
    v0 = vertices[0]
    V = v0.shape[0]
    vpad = _pad_lanes(v0, 16)
    vt0 = _pad_lanes(v0, 128)

    ni0 = _topk(v0, v0, NBR + 1)[:, 1:]
    idx0 = ni0.reshape(-1)
    nb0 = _sc_gather(vt0, idx0)

    fm0 = _conv_surface(vpad, nb0, _sd(dir0), 400)

    fo1 = _matmul_bias(fm0, w1, b1, 400)
    fs1 = _sc_gather(fo1[:, 128:], idx0)
    y1, s1a, s1b = _conv_layer(vpad, nb0, _sd(dir1), fs1, fo1[:, :128],
                               400, V)
    sc1, sh1 = _bn_coeffs(s1a, s1b, V, g1, be1)
    fm1 = _affine_relu(y1, sc1, sh1, 400)

    V1, V1p = V // 4, 2560
    samp1 = jax.random.permutation(jax.random.key(101), V)[:V1]
    pid1 = _pad_rows(ni0[samp1, :4].reshape(-1), V1p * 4, 0)
    fmp1 = _group_max(_sc_gather(fm1, pid1), 4, 128)
    vp1t = _pad_rows(_sc_gather(vt0, samp1), V1p)
    vp1 = vp1t[:, :16]
    v1r = vp1t[:V1, :3]

    ni1 = _topk(v1r, v1r, NBR + 1)[:, 1:]
    idx1 = _pad_rows(ni1.reshape(-1), V1p * NBR, 0)
    nb1 = _sc_gather(vp1t, idx1)

    fo2 = _matmul_bias(fmp1, w2, b2, 128)
    fs2 = _sc_gather(fo2[:, 256:], idx1)
    y2, s2a, s2b = _conv_layer(vp1, nb1, _sd(dir2), fs2, fo2[:, :256],
                               128, V1)
    sc2, sh2 = _bn_coeffs(s2a, s2b, V1, g2, be2)
    fm2 = _affine_relu(y2, sc2, sh2, 128)

    fo3 = _matmul_bias(fm2, w3, b3, 128)
    fs3 = _sc_gather(fo3[:, 256:], idx1)
    y3, s3a, s3b = _conv_layer(vp1, nb1, _sd(dir3), fs3, fo3[:, :256],
                               128, V1)
    sc3, sh3 = _bn_coeffs(s3a, s3b, V1, g3, be3)
    fm3 = _affine_relu(y3, sc3, sh3, 128)

    V2, V2p = V1 // 4, 640
    samp2 = jax.random.permutation(jax.random.key(202), V1)[:V2]
    pid2 = _pad_rows(ni1[samp2, :4].reshape(-1), V2p * 4, 0)
    fmp2 = _group_max(_sc_gather(fm3, pid2), 4, 128)
    vp2t = _pad_rows(_sc_gather(vp1t, samp2), V2p)
    vp2 = vp2t[:, :16]
    v2r = vp2t[:V2, :3]

    ni2 = _topk(v2r, v2r, NBR + 1)[:, 1:]
    idx2 = _pad_rows(ni2.reshape(-1), V2p * NBR, 0)
    nb2 = _sc_gather(vp2t, idx2)

    fo4 = _matmul_bias(fmp2, w4, b4, 128)
    fs4 = _sc_gather(fo4[:, 512:], idx2)
    y4, _, _ = _conv_layer(vp2, nb2, _sd(dir4), fs4, fo4[:, :512],
                           128, V2)
    fm4 = y4[:V2]

    np1 = _topk(v0, v1r, 1, nearest=True).reshape(-1)
    np2 = _topk(v0, v2r, 1, nearest=True).reshape(-1)
    fm2u = _sc_gather(fm2, np1)
    fm3u = _sc_gather(fm3, np1)
    fm4u = _sc_gather(fm4, np2)

    out = jnp.concatenate([fm0, fm1, fm2u, fm3u, fm4u], axis=1)
    return out[None]


# speedup vs baseline: 1.3362x; 1.3362x over previous
import functools

import jax
import jax.numpy as jnp
import numpy as np
from jax import lax
from jax.experimental import pallas as pl
from jax.experimental.pallas import tpu as pltpu
from jax.experimental.pallas import tpu_sc as plsc

SN = 2
NBR = 16
_NC, _NS = 2, 16
_NW = _NC * _NS


def _rup(n, m):
    return ((n + m - 1) // m) * m


def _pad_rows(x, n, fill=0.0):
    if x.shape[0] == n:
        return x
    pad = jnp.full((n - x.shape[0],) + x.shape[1:], fill, x.dtype)
    return jnp.concatenate([x, pad], axis=0)


def _pad_lanes(x, n):
    return jnp.concatenate(
        [x, jnp.zeros((x.shape[0], n - x.shape[1]), x.dtype)], axis=1)


def _topk_body(K, nearest, t_ref, st_ref, o_ref):
    t = t_ref[...]
    st = st_ref[...]
    S = st.shape[1]
    inner = jnp.dot(t, st)
    q_t = jnp.sum(t * t, axis=1, keepdims=True)
    q_s = jnp.sum(st * st, axis=0, keepdims=True)
    if nearest:
        dist = (q_s + q_t) - 2.0 * inner
    else:
        dist = (-2.0 * inner + q_s) + q_t
    iota = lax.broadcasted_iota(jnp.int32, dist.shape, 1)
    cols = []
    for _ in range(K):
        m = jnp.min(dist, axis=1, keepdims=True)
        am = jnp.min(jnp.where(dist == m, iota, S), axis=1, keepdims=True)
        cols.append(am)
        dist = jnp.where(iota == am, jnp.inf, dist)
    o_ref[...] = jnp.concatenate(cols, axis=1)


def _topk(targets, sources, K, nearest=False, Tb=256):
    T0 = targets.shape[0]
    S = sources.shape[0]
    T = _rup(T0, Tb)
    tgt = _pad_rows(_pad_lanes(targets, 16), T, 1e30)
    st = _pad_lanes(sources, 16).T
    out = pl.pallas_call(
        functools.partial(_topk_body, K, nearest),
        grid=(T // Tb,),
        in_specs=[
            pl.BlockSpec((Tb, 16), lambda i: (i, 0)),
            pl.BlockSpec((16, S), lambda i: (0, 0)),
        ],
        out_specs=pl.BlockSpec((Tb, K), lambda i: (i, 0)),
        out_shape=jax.ShapeDtypeStruct((T, K), jnp.int32),
    )(tgt, st)
    return out[:T0]


def _sc_gather_padded(table, idx, chunk):
    V, D = table.shape
    B = idx.shape[0]
    bpw = B // _NW
    n_chunks = bpw // chunk
    mesh = plsc.VectorSubcoreMesh(core_axis_name="c", subcore_axis_name="s")

    @functools.partial(
        pl.kernel,
        mesh=mesh,
        out_type=jax.ShapeDtypeStruct((B, D), jnp.float32),
        scratch_types=[
            pltpu.VMEM((bpw,), jnp.int32),
            pltpu.VMEM((chunk, D), jnp.float32),
            pltpu.SemaphoreType.DMA,
        ],
    )
    def k(table_hbm, idx_hbm, out_hbm, idx_v, rows_v, sem):
        wid = lax.axis_index("s") * _NC + lax.axis_index("c")
        base = wid * bpw
        pltpu.sync_copy(idx_hbm.at[pl.ds(base, bpw)], idx_v)

        def body(i, carry):
            pltpu.async_copy(
                table_hbm.at[idx_v.at[pl.ds(i * chunk, chunk)]], rows_v, sem
            ).wait()
            pltpu.sync_copy(rows_v, out_hbm.at[pl.ds(base + i * chunk, chunk)])
            return carry

        lax.fori_loop(0, n_chunks, body, 0)

    return k(table, idx)


def _sc_gather(table, idx):
    V, D = table.shape
    B0 = idx.shape[0]
    chunk = max(8, min(128, (110_000 // D) // 8 * 8))
    quantum = _NW * chunk
    B = _rup(B0, quantum)
    idx = _pad_rows(idx.astype(jnp.int32), B, 0)
    out = _sc_gather_padded(table, idx, chunk)
    return out[:B0]


def _mm_body(x_ref, w_ref, b_ref, o_ref):
    o_ref[...] = (
        jnp.dot(x_ref[...], w_ref[...], preferred_element_type=jnp.float32)
        + b_ref[...]
    )


def _matmul_bias(x, w, b, Rb):
    R, Cin = x.shape
    Cout = w.shape[1]
    return pl.pallas_call(
        _mm_body,
        grid=(R // Rb,),
        in_specs=[
            pl.BlockSpec((Rb, Cin), lambda i: (i, 0)),
            pl.BlockSpec((Cin, Cout), lambda i: (0, 0)),
            pl.BlockSpec((1, Cout), lambda i: (0, 0)),
        ],
        out_specs=pl.BlockSpec((Rb, Cout), lambda i: (i, 0)),
        out_shape=jax.ShapeDtypeStruct((R, Cout), jnp.float32),
    )(x, w, b[None, :])


def _theta(verts, nb, sd, Vb):
    diff = nb[:, :16].reshape(Vb, NBR, 16) - verts[:, None, :]
    ss = jnp.sum(diff * diff, axis=2, keepdims=True)
    nd = diff / jnp.maximum(jnp.sqrt(ss), 1e-12)
    th = jnp.dot(
        nd.reshape(Vb * NBR, 16), sd, preferred_element_type=jnp.float32
    )
    return jnp.maximum(th, 0.0)


def _convsurf_body(v_ref, nb_ref, sd_ref, o_ref):
    Vb = v_ref.shape[0]
    th = _theta(v_ref[...], nb_ref[...], sd_ref[...], Vb)
    m = jnp.max(th.reshape(Vb, NBR, th.shape[1]), axis=1)
    C = m.shape[1] // 2
    o_ref[...] = jnp.maximum(m[:, :C] + m[:, C:], 0.0)


def _conv_surface(vpad, nb, sd, Vb):
    V = vpad.shape[0]
    C = sd.shape[1] // 2
    return pl.pallas_call(
        _convsurf_body,
        grid=(V // Vb,),
        in_specs=[
            pl.BlockSpec((Vb, 16), lambda i: (i, 0)),
            pl.BlockSpec((Vb * NBR, 128), lambda i: (i, 0)),
            pl.BlockSpec((16, 2 * C), lambda i: (0, 0)),
        ],
        out_specs=pl.BlockSpec((Vb, C), lambda i: (i, 0)),
        out_shape=jax.ShapeDtypeStruct((V, C), jnp.float32),
    )(vpad, nb, sd)


def _convlayer_body(n_real, v_ref, nb_ref, sd_ref, fs_ref, fc_ref,
                    o_ref, s1_ref, s2_ref):
    Vb = v_ref.shape[0]
    th = _theta(v_ref[...], nb_ref[...], sd_ref[...], Vb)
    act = th * fs_ref[...]
    m = jnp.max(act.reshape(Vb, NBR, act.shape[1]), axis=1)
    C = m.shape[1] // 2
    y = fc_ref[...] + m[:, :C] + m[:, C:]
    o_ref[...] = y
    row = pl.program_id(0) * Vb + lax.broadcasted_iota(jnp.int32, (Vb, 1), 0)
    ym = jnp.where(row < n_real, y, 0.0)
    s1_ref[...] = jnp.sum(ym, axis=0).reshape(1, 1, C)
    s2_ref[...] = jnp.sum(ym * ym, axis=0).reshape(1, 1, C)


def _conv_layer(vpad, nb, sd, fs, fc, Vb, n_real):
    V = vpad.shape[0]
    C = fc.shape[1]
    nblk = V // Vb
    return pl.pallas_call(
        functools.partial(_convlayer_body, n_real),
        grid=(nblk,),
        in_specs=[
            pl.BlockSpec((Vb, 16), lambda i: (i, 0)),
            pl.BlockSpec((Vb * NBR, 128), lambda i: (i, 0)),
            pl.BlockSpec((16, 2 * C), lambda i: (0, 0)),
            pl.BlockSpec((Vb * NBR, 2 * C), lambda i: (i, 0)),
            pl.BlockSpec((Vb, C), lambda i: (i, 0)),
        ],
        out_specs=[
            pl.BlockSpec((Vb, C), lambda i: (i, 0)),
            pl.BlockSpec((1, 1, C), lambda i: (i, 0, 0)),
            pl.BlockSpec((1, 1, C), lambda i: (i, 0, 0)),
        ],
        out_shape=[
            jax.ShapeDtypeStruct((V, C), jnp.float32),
            jax.ShapeDtypeStruct((nblk, 1, C), jnp.float32),
            jax.ShapeDtypeStruct((nblk, 1, C), jnp.float32),
        ],
    )(vpad, nb, sd, fs, fc)


def _affine_body(x_ref, sc_ref, sh_ref, o_ref):
    o_ref[...] = jnp.maximum(x_ref[...] * sc_ref[...] + sh_ref[...], 0.0)


def _affine_relu(x, scale, shift, Rb):
    R, C = x.shape
    return pl.pallas_call(
        _affine_body,
        grid=(R // Rb,),
        in_specs=[
            pl.BlockSpec((Rb, C), lambda i: (i, 0)),
            pl.BlockSpec((1, C), lambda i: (0, 0)),
            pl.BlockSpec((1, C), lambda i: (0, 0)),
        ],
        out_specs=pl.BlockSpec((Rb, C), lambda i: (i, 0)),
        out_shape=jax.ShapeDtypeStruct((R, C), jnp.float32),
    )(x, scale[None, :], shift[None, :])


def _gmax_body(G, x_ref, o_ref):
    Vb = o_ref.shape[0]
    C = o_ref.shape[1]
    o_ref[...] = jnp.max(x_ref[...].reshape(Vb, G, C), axis=1)


def _group_max(x, G, Rb):
    RG, C = x.shape
    R = RG // G
    return pl.pallas_call(
        functools.partial(_gmax_body, G),
        grid=(R // Rb,),
        in_specs=[pl.BlockSpec((Rb * G, C), lambda i: (i, 0))],
        out_specs=pl.BlockSpec((Rb, C), lambda i: (i, 0)),
        out_shape=jax.ShapeDtypeStruct((R, C), jnp.float32),
    )(x)


def _sd(directions):
    n = jnp.linalg.norm(directions, axis=0, keepdims=True)
    sd = directions / jnp.maximum(n, 1e-12)
    return jnp.concatenate([sd, jnp.zeros((13, sd.shape[1]), sd.dtype)], axis=0)


def _bn_coeffs(s1, s2, n, gamma, beta, eps=1e-5):
    mean = jnp.sum(s1, axis=(0, 1)) / n
    var = jnp.sum(s2, axis=(0, 1)) / n - mean * mean
    scale = gamma / jnp.sqrt(var + eps)
    shift = beta - mean * scale
    return scale, shift


def kernel(vertices, dir0, w1, b1, dir1, w2, b2, dir2, w3, b3, dir3,
           w4, b4, dir4, g1, be1, g2, be2, g3, be3):
    v0 = vertices[0]
    V = v0.shape[0]
    vpad = _pad_lanes(v0, 16)
    vt0 = _pad_lanes(v0, 128)

    ni0 = _topk(v0, v0, NBR + 1)[:, 1:]
    idx0 = ni0.reshape(-1)
    nb0 = _sc_gather(vt0, idx0)

    fm0 = _conv_surface(vpad, nb0, _sd(dir0), 400)

    fo1 = _matmul_bias(fm0, w1, b1, 400)
    fs1 = _sc_gather(fo1[:, 128:], idx0)
    y1, s1a, s1b = _conv_layer(vpad, nb0, _sd(dir1), fs1, fo1[:, :128],
                               400, V)
    sc1, sh1 = _bn_coeffs(s1a, s1b, V, g1, be1)
    fm1 = _affine_relu(y1, sc1, sh1, 400)

    V1, V1p = V // 4, 2560
    samp1 = jax.random.permutation(jax.random.key(101), V)[:V1]
    pid1 = _pad_rows(ni0[samp1, :4].reshape(-1), V1p * 4, 0)
    fmp1 = _group_max(_sc_gather(fm1, pid1), 4, 128)
    vp1t = _pad_rows(_sc_gather(vt0, samp1), V1p)
    vp1 = vp1t[:, :16]
    v1r = vp1t[:V1, :3]

    ni1 = _topk(v1r, v1r, NBR + 1)[:, 1:]
    idx1 = _pad_rows(ni1.reshape(-1), V1p * NBR, 0)
    nb1 = _sc_gather(vp1t, idx1)

    fo2 = _matmul_bias(fmp1, w2, b2, 128)
    fs2 = _sc_gather(fo2[:, 256:], idx1)
    y2, s2a, s2b = _conv_layer(vp1, nb1, _sd(dir2), fs2, fo2[:, :256],
                               128, V1)
    sc2, sh2 = _bn_coeffs(s2a, s2b, V1, g2, be2)
    fm2 = _affine_relu(y2, sc2, sh2, 128)

    fo3 = _matmul_bias(fm2, w3, b3, 128)
    fs3 = _sc_gather(fo3[:, 256:], idx1)
    y3, s3a, s3b = _conv_layer(vp1, nb1, _sd(dir3), fs3, fo3[:, :256],
                               128, V1)
    sc3, sh3 = _bn_coeffs(s3a, s3b, V1, g3, be3)
    fm3 = _affine_relu(y3, sc3, sh3, 128)

    V2, V2p = V1 // 4, 640
    samp2 = jax.random.permutation(jax.random.key(202), V1)[:V2]
    pid2 = _pad_rows(ni1[samp2, :4].reshape(-1), V2p * 4, 0)
    fmp2 = _group_max(_sc_gather(fm3, pid2), 4, 128)
    vp2t = _pad_rows(_sc_gather(vp1t, samp2), V2p)
    vp2 = vp2t[:, :16]
    v2r = vp2t[:V2, :3]

    ni2 = _topk(v2r, v2r, NBR + 1)[:, 1:]
    idx2 = _pad_rows(ni2.reshape(-1), V2p * NBR, 0)
    nb2 = _sc_gather(vp2t, idx2)

    fo4 = _matmul_bias(fmp2, w4, b4, 128)
    fs4 = _sc_gather(fo4[:, 512:], idx2)
    y4, _, _ = _conv_layer(vp2, nb2, _sd(dir4), fs4, fo4[:, :512],
                           128, V2)
    fm4 = y4[:V2]

    np1 = _topk(v0, v1r, 1, nearest=True).reshape(-1)
    np2 = _topk(v0, v2r, 1, nearest=True).reshape(-1)
    fm2u = _sc_gather(fm2, np1)
    fm3u = _sc_gather(fm3, np1)
    fm4u = _sc_gather(fm4, np2)

    out = jnp.concatenate([fm0, fm1, fm2u, fm3u, fm4u], axis=1)
    return out[None]


# speedup vs baseline: 1.3702x; 1.0254x over previous
import functools

import jax
import jax.numpy as jnp
import numpy as np
from jax import lax
from jax.experimental import pallas as pl
from jax.experimental.pallas import tpu as pltpu
from jax.experimental.pallas import tpu_sc as plsc

SN = 2
NBR = 16
_NC, _NS = 2, 16
_NW = _NC * _NS


def _rup(n, m):
    return ((n + m - 1) // m) * m


def _pad_rows(x, n, fill=0.0):
    if x.shape[0] == n:
        return x
    pad = jnp.full((n - x.shape[0],) + x.shape[1:], fill, x.dtype)
    return jnp.concatenate([x, pad], axis=0)


def _pad_lanes(x, n):
    return jnp.concatenate(
        [x, jnp.zeros((x.shape[0], n - x.shape[1]), x.dtype)], axis=1)


def _topk_body(K, nearest, t_ref, st_ref, o_ref):
    t = t_ref[...]
    st = st_ref[...]
    S = st.shape[1]
    inner = jnp.dot(t, st)
    q_t = jnp.sum(t * t, axis=1, keepdims=True)
    q_s = jnp.sum(st * st, axis=0, keepdims=True)
    if nearest:
        dist = (q_s + q_t) - 2.0 * inner
    else:
        dist = (-2.0 * inner + q_s) + q_t
    iota = lax.broadcasted_iota(jnp.int32, dist.shape, 1)
    cols = []
    for _ in range(K):
        m = jnp.min(dist, axis=1, keepdims=True)
        am = jnp.min(jnp.where(dist == m, iota, S), axis=1, keepdims=True)
        cols.append(am)
        dist = jnp.where(iota == am, jnp.inf, dist)
    o_ref[...] = jnp.concatenate(cols, axis=1)


def _topk(targets, sources, K, nearest=False, Tb=256):
    T0 = targets.shape[0]
    S = sources.shape[0]
    T = _rup(T0, Tb)
    tgt = _pad_rows(_pad_lanes(targets, 16), T, 1e30)
    st = _pad_lanes(sources, 16).T
    out = pl.pallas_call(
        functools.partial(_topk_body, K, nearest),
        grid=(T // Tb,),
        in_specs=[
            pl.BlockSpec((Tb, 16), lambda i: (i, 0)),
            pl.BlockSpec((16, S), lambda i: (0, 0)),
        ],
        out_specs=pl.BlockSpec((Tb, K), lambda i: (i, 0)),
        out_shape=jax.ShapeDtypeStruct((T, K), jnp.int32),
    )(tgt, st)
    return out[:T0]


def _sc_gather_padded(table, idx, chunk):
    V, D = table.shape
    B = idx.shape[0]
    bpw = B // _NW
    n_chunks = bpw // chunk
    mesh = plsc.VectorSubcoreMesh(core_axis_name="c", subcore_axis_name="s")

    @functools.partial(
        pl.kernel,
        mesh=mesh,
        out_type=jax.ShapeDtypeStruct((B, D), jnp.float32),
        scratch_types=[
            pltpu.VMEM((bpw,), jnp.int32),
            pltpu.VMEM((chunk, D), jnp.float32),
            pltpu.SemaphoreType.DMA,
        ],
    )
    def k(table_hbm, idx_hbm, out_hbm, idx_v, rows_v, sem):
        wid = lax.axis_index("s") * _NC + lax.axis_index("c")
        base = wid * bpw
        pltpu.sync_copy(idx_hbm.at[pl.ds(base, bpw)], idx_v)

        def body(i, carry):
            pltpu.async_copy(
                table_hbm.at[idx_v.at[pl.ds(i * chunk, chunk)]], rows_v, sem
            ).wait()
            pltpu.sync_copy(rows_v, out_hbm.at[pl.ds(base + i * chunk, chunk)])
            return carry

        lax.fori_loop(0, n_chunks, body, 0)

    return k(table, idx)


def _sc_gather(table, idx):
    V, D = table.shape
    B0 = idx.shape[0]
    chunk = max(8, min(128, (110_000 // D) // 8 * 8))
    quantum = _NW * chunk
    B = _rup(B0, quantum)
    idx = _pad_rows(idx.astype(jnp.int32), B, 0)
    out = _sc_gather_padded(table, idx, chunk)
    return out[:B0]


def _mm_body(x_ref, w_ref, b_ref, o_ref):
    o_ref[...] = (
        jnp.dot(x_ref[...], w_ref[...], preferred_element_type=jnp.float32)
        + b_ref[...]
    )


def _matmul_bias(x, w, b, Rb):
    R, Cin = x.shape
    Cout = w.shape[1]
    return pl.pallas_call(
        _mm_body,
        grid=(R // Rb,),
        in_specs=[
            pl.BlockSpec((Rb, Cin), lambda i: (i, 0)),
            pl.BlockSpec((Cin, Cout), lambda i: (0, 0)),
            pl.BlockSpec((1, Cout), lambda i: (0, 0)),
        ],
        out_specs=pl.BlockSpec((Rb, Cout), lambda i: (i, 0)),
        out_shape=jax.ShapeDtypeStruct((R, Cout), jnp.float32),
    )(x, w, b[None, :])


def _theta(verts, nb, sd, Vb):
    diff = nb[:, :16].reshape(Vb, NBR, 16) - verts[:, None, :]
    ss = jnp.sum(diff * diff, axis=2, keepdims=True)
    nd = diff / jnp.maximum(jnp.sqrt(ss), 1e-12)
    th = jnp.dot(
        nd.reshape(Vb * NBR, 16), sd, preferred_element_type=jnp.float32
    )
    return jnp.maximum(th, 0.0)


def _convsurf_body(v_ref, nb_ref, sd_ref, o_ref):
    Vb = v_ref.shape[0]
    th = _theta(v_ref[...], nb_ref[...], sd_ref[...], Vb)
    m = jnp.max(th.reshape(Vb, NBR, th.shape[1]), axis=1)
    C = m.shape[1] // 2
    o_ref[...] = jnp.maximum(m[:, :C] + m[:, C:], 0.0)


def _conv_surface(vpad, nb, sd, Vb):
    V = vpad.shape[0]
    C = sd.shape[1] // 2
    return pl.pallas_call(
        _convsurf_body,
        grid=(V // Vb,),
        in_specs=[
            pl.BlockSpec((Vb, 16), lambda i: (i, 0)),
            pl.BlockSpec((Vb * NBR, 128), lambda i: (i, 0)),
            pl.BlockSpec((16, 2 * C), lambda i: (0, 0)),
        ],
        out_specs=pl.BlockSpec((Vb, C), lambda i: (i, 0)),
        out_shape=jax.ShapeDtypeStruct((V, C), jnp.float32),
    )(vpad, nb, sd)


def _convlayer_body(n_real, v_ref, nb_ref, sd_ref, fs_ref, fc_ref,
                    o_ref, s1_ref, s2_ref):
    Vb = v_ref.shape[0]
    th = _theta(v_ref[...], nb_ref[...], sd_ref[...], Vb)
    act = th * fs_ref[...]
    m = jnp.max(act.reshape(Vb, NBR, act.shape[1]), axis=1)
    C = m.shape[1] // 2
    y = fc_ref[...] + m[:, :C] + m[:, C:]
    o_ref[...] = y
    row = pl.program_id(0) * Vb + lax.broadcasted_iota(jnp.int32, (Vb, 1), 0)
    ym = jnp.where(row < n_real, y, 0.0)
    s1_ref[...] = jnp.sum(ym, axis=0).reshape(1, 1, C)
    s2_ref[...] = jnp.sum(ym * ym, axis=0).reshape(1, 1, C)


def _conv_layer(vpad, nb, sd, fs, fc, Vb, n_real):
    V = vpad.shape[0]
    C = fc.shape[1]
    nblk = V // Vb
    return pl.pallas_call(
        functools.partial(_convlayer_body, n_real),
        grid=(nblk,),
        in_specs=[
            pl.BlockSpec((Vb, 16), lambda i: (i, 0)),
            pl.BlockSpec((Vb * NBR, 128), lambda i: (i, 0)),
            pl.BlockSpec((16, 2 * C), lambda i: (0, 0)),
            pl.BlockSpec((Vb * NBR, 2 * C), lambda i: (i, 0)),
            pl.BlockSpec((Vb, C), lambda i: (i, 0)),
        ],
        out_specs=[
            pl.BlockSpec((Vb, C), lambda i: (i, 0)),
            pl.BlockSpec((1, 1, C), lambda i: (i, 0, 0)),
            pl.BlockSpec((1, 1, C), lambda i: (i, 0, 0)),
        ],
        out_shape=[
            jax.ShapeDtypeStruct((V, C), jnp.float32),
            jax.ShapeDtypeStruct((nblk, 1, C), jnp.float32),
            jax.ShapeDtypeStruct((nblk, 1, C), jnp.float32),
        ],
    )(vpad, nb, sd, fs, fc)


def _affine_body(x_ref, sc_ref, sh_ref, o_ref):
    o_ref[...] = jnp.maximum(x_ref[...] * sc_ref[...] + sh_ref[...], 0.0)


def _affine_relu(x, scale, shift, Rb):
    R, C = x.shape
    return pl.pallas_call(
        _affine_body,
        grid=(R // Rb,),
        in_specs=[
            pl.BlockSpec((Rb, C), lambda i: (i, 0)),
            pl.BlockSpec((1, C), lambda i: (0, 0)),
            pl.BlockSpec((1, C), lambda i: (0, 0)),
        ],
        out_specs=pl.BlockSpec((Rb, C), lambda i: (i, 0)),
        out_shape=jax.ShapeDtypeStruct((R, C), jnp.float32),
    )(x, scale[None, :], shift[None, :])


def _gmax_body(G, x_ref, o_ref):
    Vb = o_ref.shape[0]
    C = o_ref.shape[1]
    o_ref[...] = jnp.max(x_ref[...].reshape(Vb, G, C), axis=1)


def _group_max(x, G, Rb):
    RG, C = x.shape
    R = RG // G
    return pl.pallas_call(
        functools.partial(_gmax_body, G),
        grid=(R // Rb,),
        in_specs=[pl.BlockSpec((Rb * G, C), lambda i: (i, 0))],
        out_specs=pl.BlockSpec((Rb, C), lambda i: (i, 0)),
        out_shape=jax.ShapeDtypeStruct((R, C), jnp.float32),
    )(x)


def _sd(directions):
    n = jnp.linalg.norm(directions, axis=0, keepdims=True)
    sd = directions / jnp.maximum(n, 1e-12)
    return jnp.concatenate([sd, jnp.zeros((13, sd.shape[1]), sd.dtype)], axis=0)


def _bn_coeffs(s1, s2, n, gamma, beta, eps=1e-5):
    mean = jnp.sum(s1, axis=(0, 1)) / n
    var = jnp.sum(s2, axis=(0, 1)) / n - mean * mean
    scale = gamma / jnp.sqrt(var + eps)
    shift = beta - mean * scale
    return scale, shift


def kernel(vertices, dir0, w1, b1, dir1, w2, b2, dir2, w3, b3, dir3,
           w4, b4, dir4, g1, be1, g2, be2, g3, be3):
    v0 = vertices[0]
    V = v0.shape[0]
    vpad = _pad_lanes(v0, 16)
    vt0 = _pad_lanes(v0, 128)

    V1, V1p = V // 4, 2560
    samp1 = jax.random.permutation(jax.random.key(101), V)[:V1]
    ni0 = _topk(v0, v0, NBR + 1)[:, 1:]
    idx0 = ni0.reshape(-1)
    g0 = _sc_gather(vt0, jnp.concatenate([idx0, samp1]))
    nb0 = g0[: V * NBR]

    fm0 = _conv_surface(vpad, nb0, _sd(dir0), 400)

    fo1 = _matmul_bias(fm0, w1, b1, 400)
    fs1 = _sc_gather(fo1[:, 128:], idx0)
    y1, s1a, s1b = _conv_layer(vpad, nb0, _sd(dir1), fs1, fo1[:, :128],
                               400, V)
    sc1, sh1 = _bn_coeffs(s1a, s1b, V, g1, be1)
    fm1 = _affine_relu(y1, sc1, sh1, 400)

    pid1 = _pad_rows(ni0[samp1, :4].reshape(-1), V1p * 4, 0)
    fmp1 = _group_max(_sc_gather(fm1, pid1), 4, 128)
    vp1t = _pad_rows(g0[V * NBR:V * NBR + V1], V1p)
    vp1 = vp1t[:, :16]
    v1r = vp1t[:V1, :3]

    V2, V2p = V1 // 4, 640
    samp2 = jax.random.permutation(jax.random.key(202), V1)[:V2]
    ni1 = _topk(v1r, v1r, NBR + 1)[:, 1:]
    idx1 = _pad_rows(ni1.reshape(-1), V1p * NBR, 0)
    g1 = _sc_gather(vp1t, jnp.concatenate([idx1, samp2]))
    nb1 = g1[: V1p * NBR]

    fo2 = _matmul_bias(fmp1, w2, b2, 128)
    fs2 = _sc_gather(fo2[:, 256:], idx1)
    y2, s2a, s2b = _conv_layer(vp1, nb1, _sd(dir2), fs2, fo2[:, :256],
                               128, V1)
    sc2, sh2 = _bn_coeffs(s2a, s2b, V1, g2, be2)
    fm2 = _affine_relu(y2, sc2, sh2, 128)

    fo3 = _matmul_bias(fm2, w3, b3, 128)
    fs3 = _sc_gather(fo3[:, 256:], idx1)
    y3, s3a, s3b = _conv_layer(vp1, nb1, _sd(dir3), fs3, fo3[:, :256],
                               128, V1)
    sc3, sh3 = _bn_coeffs(s3a, s3b, V1, g3, be3)
    fm3 = _affine_relu(y3, sc3, sh3, 128)

    np1 = _topk(v0, v1r, 1, nearest=True).reshape(-1)
    pid2 = _pad_rows(ni1[samp2, :4].reshape(-1), V2p * 4, 0)
    g3 = _sc_gather(fm3, jnp.concatenate([pid2, np1]))
    fmp2 = _group_max(g3[: V2p * 4], 4, 128)
    fm3u = g3[V2p * 4:]
    vp2t = _pad_rows(g1[V1p * NBR:V1p * NBR + V2], V2p)
    vp2 = vp2t[:, :16]
    v2r = vp2t[:V2, :3]

    ni2 = _topk(v2r, v2r, NBR + 1)[:, 1:]
    idx2 = _pad_rows(ni2.reshape(-1), V2p * NBR, 0)
    nb2 = _sc_gather(vp2t, idx2)

    fo4 = _matmul_bias(fmp2, w4, b4, 128)
    fs4 = _sc_gather(fo4[:, 512:], idx2)
    y4, _, _ = _conv_layer(vp2, nb2, _sd(dir4), fs4, fo4[:, :512],
                           128, V2)
    fm4 = y4[:V2]

    np2 = _topk(v0, v2r, 1, nearest=True).reshape(-1)
    fm2u = _sc_gather(fm2, np1)
    fm4u = _sc_gather(fm4, np2)

    out = jnp.concatenate([fm0, fm1, fm2u, fm3u, fm4u], axis=1)
    return out[None]
